# baseline (device time: 6746 ns/iter reference)
import jax
import jax.numpy as jnp
from jax import lax
from jax.experimental import pallas as pl
from jax.experimental.pallas import tpu as pltpu

CHUNKS = 2


def kernel(x):
    _, m, n = x.shape
    half = n // 2
    rows = m // CHUNKS

    def body(
        x_hbm,
        out_hbm,
        pf32,
        mine,
        send,
        recv,
        out_v,
        p_sems,
        m_sem,
        send_sems,
        recv_sems,
        out_sems,
    ):
        my_x = lax.axis_index("x")
        my_y = lax.axis_index("y")
        my_z = lax.axis_index("z")
        partner_y = 1 - my_y
        partner = (my_x, partner_y, my_z)

        barrier_sem = pltpu.get_barrier_semaphore()
        pl.semaphore_signal(
            barrier_sem,
            inc=1,
            device_id=partner,
            device_id_type=pl.DeviceIdType.MESH,
        )

        p_dmas = []
        for c in range(CHUNKS):
            d = pltpu.make_async_copy(
                x_hbm.at[0, pl.ds(c * rows, rows), pl.ds(partner_y * half, half)],
                pf32.at[pl.ds(c * rows, rows), :],
                p_sems.at[c],
            )
            d.start()
            p_dmas.append(d)
        m_dma = pltpu.make_async_copy(
            x_hbm.at[0, :, pl.ds(my_y * half, half)],
            mine,
            m_sem,
        )
        m_dma.start()

        rdmas = []
        for c in range(CHUNKS):
            rs = pl.ds(c * rows, rows)
            p_dmas[c].wait()
            send[rs, :] = pf32[rs, :].astype(send.dtype)
            if c == 0:
                pl.semaphore_wait(barrier_sem, 1)
            rdma = pltpu.make_async_remote_copy(
                src_ref=send.at[rs, :],
                dst_ref=recv.at[rs, :],
                send_sem=send_sems.at[c],
                recv_sem=recv_sems.at[c],
                device_id=partner,
                device_id_type=pl.DeviceIdType.MESH,
            )
            rdma.start()
            rdmas.append(rdma)

        m_dma.wait()
        out_dmas = []
        for c in range(CHUNKS):
            rs = pl.ds(c * rows, rows)
            rdmas[c].wait_recv()
            out_v[rs, :] = (
                mine[rs, :] + recv[rs, :].astype(jnp.float32)
            ).astype(out_v.dtype)
            d = pltpu.make_async_copy(
                out_v.at[rs, :], out_hbm.at[rs, :], out_sems.at[c]
            )
            d.start()
            out_dmas.append(d)

        for c in range(CHUNKS):
            out_dmas[c].wait()
            rdmas[c].wait_send()

    return pl.pallas_call(
        body,
        out_shape=jax.ShapeDtypeStruct((m, half), jnp.bfloat16),
        in_specs=[pl.BlockSpec(memory_space=pl.MemorySpace.ANY)],
        out_specs=pl.BlockSpec(memory_space=pl.MemorySpace.ANY),
        scratch_shapes=[
            pltpu.VMEM((m, half), jnp.float32),
            pltpu.VMEM((m, half), jnp.float32),
            pltpu.VMEM((m, half), jnp.bfloat16),
            pltpu.VMEM((m, half), jnp.bfloat16),
            pltpu.VMEM((m, half), jnp.bfloat16),
            pltpu.SemaphoreType.DMA((CHUNKS,)),
            pltpu.SemaphoreType.DMA,
            pltpu.SemaphoreType.DMA((CHUNKS,)),
            pltpu.SemaphoreType.DMA((CHUNKS,)),
            pltpu.SemaphoreType.DMA((CHUNKS,)),
        ],
        compiler_params=pltpu.CompilerParams(collective_id=0),
    )(x)


# device time: 6424 ns/iter; 1.0501x vs baseline; 1.0501x over previous
import jax
import jax.numpy as jnp
from jax import lax
from jax.experimental import pallas as pl
from jax.experimental.pallas import tpu as pltpu

CHUNKS = 4


def kernel(x):
    _, m, n = x.shape
    half = n // 2
    rows = m // CHUNKS

    x = pltpu.with_memory_space_constraint(x, pltpu.MemorySpace.HBM)

    def body(
        x_hbm,
        out_ref,
        pf32,
        mine,
        send,
        recv,
        p_sems,
        m_sem,
        send_sems,
        recv_sems,
    ):
        my_x = lax.axis_index("x")
        my_y = lax.axis_index("y")
        my_z = lax.axis_index("z")
        partner_y = 1 - my_y
        partner = (my_x, partner_y, my_z)

        barrier_sem = pltpu.get_barrier_semaphore()
        pl.semaphore_signal(
            barrier_sem,
            inc=1,
            device_id=partner,
            device_id_type=pl.DeviceIdType.MESH,
        )

        p_dmas = []
        for c in range(CHUNKS):
            d = pltpu.make_async_copy(
                x_hbm.at[0, pl.ds(c * rows, rows), pl.ds(partner_y * half, half)],
                pf32.at[pl.ds(c * rows, rows), :],
                p_sems.at[c],
            )
            d.start()
            p_dmas.append(d)
        rdmas = []
        for c in range(CHUNKS):
            rs = pl.ds(c * rows, rows)
            p_dmas[c].wait()
            send[rs, :] = pf32[rs, :].astype(send.dtype)
            if c == 0:
                pl.semaphore_wait(barrier_sem, 1)
            rdma = pltpu.make_async_remote_copy(
                src_ref=send.at[rs, :],
                dst_ref=recv.at[rs, :],
                send_sem=send_sems.at[c],
                recv_sem=recv_sems.at[c],
                device_id=partner,
                device_id_type=pl.DeviceIdType.MESH,
            )
            rdma.start()
            rdmas.append(rdma)

        m_dma = pltpu.make_async_copy(
            x_hbm.at[0, :, pl.ds(my_y * half, half)],
            mine,
            m_sem,
        )
        m_dma.start()
        m_dma.wait()
        for c in range(CHUNKS):
            rs = pl.ds(c * rows, rows)
            rdmas[c].wait_recv()
            out_ref[rs, :] = (
                mine[rs, :] + recv[rs, :].astype(jnp.float32)
            ).astype(out_ref.dtype)

        for c in range(CHUNKS):
            rdmas[c].wait_send()

    return pl.pallas_call(
        body,
        out_shape=jax.ShapeDtypeStruct((m, half), jnp.bfloat16),
        in_specs=[pl.BlockSpec(memory_space=pltpu.MemorySpace.HBM)],
        out_specs=pl.BlockSpec(memory_space=pltpu.MemorySpace.VMEM),
        scratch_shapes=[
            pltpu.VMEM((m, half), jnp.float32),
            pltpu.VMEM((m, half), jnp.float32),
            pltpu.VMEM((m, half), jnp.bfloat16),
            pltpu.VMEM((m, half), jnp.bfloat16),
            pltpu.SemaphoreType.DMA((CHUNKS,)),
            pltpu.SemaphoreType.DMA,
            pltpu.SemaphoreType.DMA((CHUNKS,)),
            pltpu.SemaphoreType.DMA((CHUNKS,)),
        ],
        compiler_params=pltpu.CompilerParams(collective_id=0),
    )(x)


# device time: 6413 ns/iter; 1.0519x vs baseline; 1.0017x over previous
import jax
import jax.numpy as jnp
from jax import lax
from jax.experimental import pallas as pl
from jax.experimental.pallas import tpu as pltpu

CHUNKS = 2


def kernel(x):
    _, m, n = x.shape
    half = n // 2
    rows = m // CHUNKS

    x = pltpu.with_memory_space_constraint(x, pltpu.MemorySpace.HBM)

    def body(
        x_hbm,
        out_ref,
        pf32,
        mine,
        send,
        recv,
        p_sems,
        m_sem,
        send_sems,
        recv_sems,
    ):
        my_x = lax.axis_index("x")
        my_y = lax.axis_index("y")
        my_z = lax.axis_index("z")
        partner_y = 1 - my_y
        partner = (my_x, partner_y, my_z)

        barrier_sem = pltpu.get_barrier_semaphore()
        pl.semaphore_signal(
            barrier_sem,
            inc=1,
            device_id=partner,
            device_id_type=pl.DeviceIdType.MESH,
        )

        p_dmas = []
        for c in range(CHUNKS):
            d = pltpu.make_async_copy(
                x_hbm.at[0, pl.ds(c * rows, rows), pl.ds(partner_y * half, half)],
                pf32.at[pl.ds(c * rows, rows), :],
                p_sems.at[c],
            )
            d.start()
            p_dmas.append(d)
        rdmas = []
        for c in range(CHUNKS):
            rs = pl.ds(c * rows, rows)
            p_dmas[c].wait()
            send[rs, :] = pf32[rs, :].astype(send.dtype)
            if c == 0:
                pl.semaphore_wait(barrier_sem, 1)
            rdma = pltpu.make_async_remote_copy(
                src_ref=send.at[rs, :],
                dst_ref=recv.at[rs, :],
                send_sem=send_sems.at[c],
                recv_sem=recv_sems.at[c],
                device_id=partner,
                device_id_type=pl.DeviceIdType.MESH,
            )
            rdma.start()
            rdmas.append(rdma)

        m_dma = pltpu.make_async_copy(
            x_hbm.at[0, :, pl.ds(my_y * half, half)],
            mine,
            m_sem,
        )
        m_dma.start()
        m_dma.wait()
        for c in range(CHUNKS):
            rs = pl.ds(c * rows, rows)
            rdmas[c].wait_recv()
            out_ref[rs, :] = (
                mine[rs, :] + recv[rs, :].astype(jnp.float32)
            ).astype(out_ref.dtype)

        for c in range(CHUNKS):
            rdmas[c].wait_send()

    return pl.pallas_call(
        body,
        out_shape=jax.ShapeDtypeStruct((m, half), jnp.bfloat16),
        in_specs=[pl.BlockSpec(memory_space=pltpu.MemorySpace.HBM)],
        out_specs=pl.BlockSpec(memory_space=pltpu.MemorySpace.VMEM),
        scratch_shapes=[
            pltpu.VMEM((m, half), jnp.float32),
            pltpu.VMEM((m, half), jnp.float32),
            pltpu.VMEM((m, half), jnp.bfloat16),
            pltpu.VMEM((m, half), jnp.bfloat16),
            pltpu.SemaphoreType.DMA((CHUNKS,)),
            pltpu.SemaphoreType.DMA,
            pltpu.SemaphoreType.DMA((CHUNKS,)),
            pltpu.SemaphoreType.DMA((CHUNKS,)),
        ],
        compiler_params=pltpu.CompilerParams(collective_id=0),
    )(x)
